# baseline (speedup 1.0000x reference)
import functools

import jax
import jax.numpy as jnp
from jax import lax
from jax.experimental import pallas as pl
from jax.experimental.pallas import tpu as pltpu
from jax.experimental.pallas import tpu_sc as plsc

EMBED = 128
NC = 2
NS = 16
NW = NC * NS
G = 64
NB = 8
IC = 10
NA = 3
L = 16


def _sc_gather(xf2, table):
    rows_total, g = xf2.shape
    assert g == G
    B = rows_total * G
    per_w_rows = rows_total // NW
    nb_iter = per_w_rows // NB
    nchunk = per_w_rows // (NB * IC)
    assert per_w_rows % (NB * IC) == 0
    mesh = plsc.VectorSubcoreMesh(core_axis_name="c", subcore_axis_name="s")

    scratch = [
        pltpu.VMEM((2, IC * NB, G), jnp.int32),
        pltpu.VMEM((NB, G, EMBED), jnp.float32),
        pltpu.VMEM((NA, G, EMBED), jnp.float32),
        pltpu.VMEM_SHARED((32, EMBED), jnp.float32),
        pltpu.VMEM((32, EMBED), jnp.float32),
        pltpu.SemaphoreType.DMA,
    ] + [pltpu.SemaphoreType.DMA] * (2 * NB)

    @functools.partial(
        pl.kernel,
        mesh=mesh,
        out_type=jax.ShapeDtypeStruct((B, EMBED), jnp.float32),
        scratch_types=scratch,
        compiler_params=pltpu.CompilerParams(needs_layout_passes=False),
    )
    def k(idx_hbm, table_hbm, out_hbm, idx_v, rows, arows,
          table_sp, table_tile, i_sem, *sems):
        g_sems = sems[:NB]
        st_sems = sems[NB:]
        wid = lax.axis_index("s") * NC + lax.axis_index("c")
        base_row = wid * per_w_rows
        CH = IC * NB

        @pl.when(lax.axis_index("s") == 0)
        def _load_table():
            pltpu.sync_copy(table_hbm, table_sp)

        pltpu.sync_copy(table_hbm, table_tile)
        pltpu.async_copy(idx_hbm.at[pl.ds(base_row, CH)], idx_v.at[0], i_sem)
        plsc.subcore_barrier()

        def body(j, carry):
            row0 = base_row + j * NB
            t = lax.div(j, IC)
            par = lax.rem(t, 2)

            @pl.when(lax.rem(j, IC) == 0)
            def _stage():
                pltpu.make_async_copy(
                    idx_hbm.at[pl.ds(pl.multiple_of(base_row + t * CH, 8), CH)],
                    idx_v.at[par], i_sem).wait()

                @pl.when(t + 1 < nchunk)
                def _prefetch():
                    pltpu.async_copy(
                        idx_hbm.at[pl.ds(
                            pl.multiple_of(base_row + (t + 1) * CH, 8), CH)],
                        idx_v.at[1 - par], i_sem)

            ib = lax.rem(j, IC) * NB
            NG = NB - NA
            for b in range(NG):
                @pl.when(j > 0)
                def _drain(b=b):
                    pltpu.make_async_copy(
                        rows.at[b], out_hbm.at[pl.ds((row0 + b) * G, G)],
                        st_sems[b]).wait()
                pltpu.async_copy(table_sp.at[idx_v.at[par, ib + b]], rows.at[b],
                                 g_sems[b])
            lane = lax.iota(jnp.int32, L)
            for b in range(NG, NB):
                a = b - NG

                @pl.when(j > 0)
                def _drain(b=b, a=a):
                    pltpu.make_async_copy(
                        arows.at[a], out_hbm.at[pl.ds((row0 + b) * G, G)],
                        st_sems[b]).wait()
                addrs = []
                for q in range(G // L):
                    iv = idx_v[par, ib + b, pl.ds(q * L, L)]
                    addrs.append((iv, q * L + lane))

                def asm_body(c8, col0, a=a, addrs=addrs):
                    for cc in range(8):
                        col = col0 + cc
                        for src_rows, dst_rows in addrs:
                            vals = plsc.load_gather(table_tile,
                                                    [src_rows, col])
                            plsc.store_scatter(arows.at[a], [dst_rows, col],
                                               vals)
                    return col0 + 8

                lax.fori_loop(0, EMBED // 8, asm_body, lane * 0)
                pltpu.async_copy(arows.at[a],
                                 out_hbm.at[pl.ds((row0 + b) * G, G)],
                                 st_sems[b])
            for b in range(NG):
                pltpu.make_async_copy(table_sp.at[idx_v.at[par, ib + b]],
                                      rows.at[b], g_sems[b]).wait()
                pltpu.async_copy(rows.at[b], out_hbm.at[pl.ds((row0 + b) * G, G)],
                                 st_sems[b])
            return carry

        lax.fori_loop(0, nb_iter, body, 0)
        row_last = base_row + (nb_iter - 1) * NB
        for b in range(NB - NA):
            pltpu.make_async_copy(
                rows.at[b], out_hbm.at[pl.ds((row_last + b) * G, G)],
                st_sems[b]).wait()
        for b in range(NB - NA, NB):
            pltpu.make_async_copy(
                arows.at[b - (NB - NA)],
                out_hbm.at[pl.ds((row_last + b) * G, G)], st_sems[b]).wait()

    return k(xf2, table)


def kernel(x, table):
    n, s = x.shape
    xf2 = x.reshape((n * s) // G, G)
    out = _sc_gather(xf2, table)
    return out.reshape(n, s, EMBED)


# speedup vs baseline: 8.7838x; 8.7838x over previous
import functools

import jax
import jax.numpy as jnp
from jax import lax
from jax.experimental import pallas as pl
from jax.experimental.pallas import tpu as pltpu
from jax.experimental.pallas import tpu_sc as plsc

EMBED = 128
NC = 2
NS = 16
NW = NC * NS
G = 64
NB = 10
IC = 8


def _sc_gather(xf2, table):
    rows_total, g = xf2.shape
    assert g == G
    B = rows_total * G
    per_w_rows = rows_total // NW
    nb_iter = per_w_rows // NB
    nchunk = per_w_rows // (NB * IC)
    assert per_w_rows % (NB * IC) == 0
    mesh = plsc.VectorSubcoreMesh(core_axis_name="c", subcore_axis_name="s")

    scratch = [
        pltpu.VMEM((2, IC * NB, G), jnp.int32),
        pltpu.VMEM((NB, G, EMBED), jnp.float32),
        pltpu.VMEM_SHARED((32, EMBED), jnp.float32),
        pltpu.SemaphoreType.DMA,
    ] + [pltpu.SemaphoreType.DMA] * (2 * NB)

    @functools.partial(
        pl.kernel,
        mesh=mesh,
        out_type=jax.ShapeDtypeStruct((B, EMBED), jnp.float32),
        scratch_types=scratch,
    )
    def k(idx_hbm, table_hbm, out_hbm, idx_v, rows, table_sp, i_sem, *sems):
        g_sems = sems[:NB]
        st_sems = sems[NB:]
        wid = lax.axis_index("s") * NC + lax.axis_index("c")
        base_row = wid * per_w_rows
        CH = IC * NB

        @pl.when(lax.axis_index("s") == 0)
        def _load_table():
            pltpu.sync_copy(table_hbm, table_sp)

        pltpu.async_copy(idx_hbm.at[pl.ds(base_row, CH)], idx_v.at[0], i_sem)
        plsc.subcore_barrier()

        def body(j, carry):
            row0 = base_row + j * NB
            t = lax.div(j, IC)
            par = lax.rem(t, 2)

            @pl.when(lax.rem(j, IC) == 0)
            def _stage():
                pltpu.make_async_copy(
                    idx_hbm.at[pl.ds(pl.multiple_of(base_row + t * CH, 8), CH)],
                    idx_v.at[par], i_sem).wait()

                @pl.when(t + 1 < nchunk)
                def _prefetch():
                    pltpu.async_copy(
                        idx_hbm.at[pl.ds(
                            pl.multiple_of(base_row + (t + 1) * CH, 8), CH)],
                        idx_v.at[1 - par], i_sem)

            ib = lax.rem(j, IC) * NB
            for b in range(NB):
                @pl.when(j > 0)
                def _drain(b=b):
                    pltpu.make_async_copy(
                        rows.at[b], out_hbm.at[pl.ds((row0 + b) * G, G)],
                        st_sems[b]).wait()
                pltpu.async_copy(table_sp.at[idx_v.at[par, ib + b]], rows.at[b],
                                 g_sems[b])
            for b in range(NB):
                pltpu.make_async_copy(table_sp.at[idx_v.at[par, ib + b]],
                                      rows.at[b], g_sems[b]).wait()
                pltpu.async_copy(rows.at[b], out_hbm.at[pl.ds((row0 + b) * G, G)],
                                 st_sems[b])
            return carry

        lax.fori_loop(0, nb_iter, body, 0)
        row_last = base_row + (nb_iter - 1) * NB
        for b in range(NB):
            pltpu.make_async_copy(
                rows.at[b], out_hbm.at[pl.ds((row_last + b) * G, G)],
                st_sems[b]).wait()

    return k(xf2, table)


def kernel(x, table):
    n, s = x.shape
    xf2 = x.reshape((n * s) // G, G)
    out = _sc_gather(xf2, table)
    return out.reshape(n, s, EMBED)
